# baseline (speedup 1.0000x reference)
import functools
import jax
import jax.numpy as jnp
from jax import lax
from jax.experimental import pallas as pl
from jax.experimental.pallas import tpu as pltpu, tpu_sc as plsc

_N = 1000
_ROWS = 51200
_NC, _NS = 2, 16
_NW = _NC * _NS
_RPW = _ROWS // _NW
_CR = 64
_NCH = _RPW // _CR
_CE = _CR * _N
_NBUF = 2


def _sc_body(idx_hbm, out_hbm, idx_v, buf0, buf1, sem0, sem1):
    wid = lax.axis_index("s") * _NC + lax.axis_index("c")
    base_row = wid * _RPW
    pltpu.sync_copy(idx_hbm.at[pl.ds(base_row, _RPW)], idx_v)

    zeros16 = jnp.zeros((16,), jnp.float32)
    ones16 = jnp.ones((16,), jnp.float32)
    lane = lax.iota(jnp.int32, 16)
    sems = (sem0, sem1)
    bufs = (buf0, buf1)

    def zbody(i, carry):
        buf0[pl.ds(i * 16, 16)] = zeros16
        buf1[pl.ds(i * 16, 16)] = zeros16
        return carry
    lax.fori_loop(0, _CE // 16, zbody, 0)

    def scatter_chunk(b, c, vals):
        for r in range(_CR // 16):
            iv = idx_v[pl.ds(c * _CR + r * 16, 16)]
            flat = (lane + r * 16) * _N + iv
            plsc.store_scatter(bufs[b], [flat], vals)

    def fire(b, c):
        dst = out_hbm.at[pl.ds((base_row + c * _CR) * _N, _CE)]
        pltpu.async_copy(bufs[b], dst, sems[b])

    def wait(b):
        pltpu.make_async_copy(
            bufs[b], out_hbm.at[pl.ds(base_row * _N, _CE)], sems[b]
        ).wait()

    for b in range(_NBUF):
        scatter_chunk(b, b, ones16)
        fire(b, b)

    def ring_body(c, carry):
        def step(b):
            wait(b)
            scatter_chunk(b, c - _NBUF, zeros16)
            scatter_chunk(b, c, ones16)
            fire(b, c)

        @pl.when(lax.rem(c, 2) == 0)
        def _():
            step(0)

        @pl.when(lax.rem(c, 2) == 1)
        def _():
            step(1)
        return carry
    lax.fori_loop(_NBUF, _NCH, ring_body, 0)

    for b in range(_NBUF):
        wait(b)


def sc_one_hot(flat_idx_i32):
    mesh = plsc.VectorSubcoreMesh(core_axis_name="c", subcore_axis_name="s")
    k = functools.partial(
        pl.kernel, mesh=mesh,
        compiler_params=pltpu.CompilerParams(needs_layout_passes=False),
        out_type=jax.ShapeDtypeStruct((_ROWS * _N,), jnp.float32),
        scratch_types=[
            pltpu.VMEM((_RPW,), jnp.int32),
            pltpu.VMEM((_CE,), jnp.float32),
            pltpu.VMEM((_CE,), jnp.float32),
            pltpu.SemaphoreType.DMA,
            pltpu.SemaphoreType.DMA,
        ],
    )(_sc_body)
    return k(flat_idx_i32)


def kernel(input, eye):
    n = eye.shape[0]
    flat = input.reshape(-1).astype(jnp.int32)
    out = sc_one_hot(flat)
    return out.reshape(*input.shape, n)


# speedup vs baseline: 1.4711x; 1.4711x over previous
import functools
import jax
import jax.numpy as jnp
from jax import lax
from jax.experimental import pallas as pl
from jax.experimental.pallas import tpu as pltpu, tpu_sc as plsc

_N = 1000
_ROWS = 51200
_NC, _NS = 2, 16
_NW = _NC * _NS
_RPW = _ROWS // _NW
_CR = 32
_NCH = _RPW // _CR
_NBUF = 2


def _sc_body(idx_hbm, zeros_hbm, out_hbm, idx_v, buf0, buf1, sem0, sem1):
    wid = lax.axis_index("s") * _NC + lax.axis_index("c")
    base_row = wid * _RPW
    pltpu.sync_copy(idx_hbm.at[pl.ds(base_row, _RPW)], idx_v)
    pltpu.sync_copy(zeros_hbm, buf0)
    pltpu.sync_copy(zeros_hbm, buf1)

    zeros16 = jnp.zeros((16,), jnp.float32)
    ones16 = jnp.ones((16,), jnp.float32)
    lane = lax.iota(jnp.int32, 16)
    sems = (sem0, sem1)
    bufs = (buf0, buf1)

    def scatter_chunk(b, c, vals):
        for r in range(_CR // 16):
            iv = idx_v[pl.ds(c * _CR + r * 16, 16)]
            rows = lane + r * 16
            plsc.store_scatter(bufs[b], [rows, iv], vals)

    def fire(b, c):
        dst = out_hbm.at[pl.ds(base_row + c * _CR, _CR)]
        pltpu.async_copy(bufs[b], dst, sems[b])

    def wait(b):
        pltpu.make_async_copy(
            bufs[b], out_hbm.at[pl.ds(base_row, _CR)], sems[b]
        ).wait()

    for b in range(_NBUF):
        scatter_chunk(b, b, ones16)
        fire(b, b)

    def ring_body(c, carry):
        def step(b):
            wait(b)
            scatter_chunk(b, c - _NBUF, zeros16)
            scatter_chunk(b, c, ones16)
            fire(b, c)

        @pl.when(lax.rem(c, 2) == 0)
        def _():
            step(0)

        @pl.when(lax.rem(c, 2) == 1)
        def _():
            step(1)
        return carry
    lax.fori_loop(_NBUF, _NCH, ring_body, 0)

    for b in range(_NBUF):
        wait(b)


def _sc_one_hot(flat_idx_i32, zeros_chunk):
    mesh = plsc.VectorSubcoreMesh(core_axis_name="c", subcore_axis_name="s")
    k = functools.partial(
        pl.kernel, mesh=mesh,
        compiler_params=pltpu.CompilerParams(needs_layout_passes=False),
        out_type=jax.ShapeDtypeStruct((_ROWS, _N), jnp.float32),
        scratch_types=[
            pltpu.VMEM((_RPW,), jnp.int32),
            pltpu.VMEM((_CR, _N), jnp.float32),
            pltpu.VMEM((_CR, _N), jnp.float32),
            pltpu.SemaphoreType.DMA,
            pltpu.SemaphoreType.DMA,
        ],
    )(_sc_body)
    return k(flat_idx_i32, zeros_chunk)


def kernel(input, eye):
    n = eye.shape[0]
    flat = input.reshape(-1).astype(jnp.int32)
    zeros_chunk = jnp.zeros((_CR, _N), jnp.float32)
    out = _sc_one_hot(flat, zeros_chunk)
    return out.reshape(*input.shape, n)


# speedup vs baseline: 1.8530x; 1.2596x over previous
import functools
import jax
import jax.numpy as jnp
from jax import lax
from jax.experimental import pallas as pl
from jax.experimental.pallas import tpu as pltpu, tpu_sc as plsc

_N = 1000
_B = 1024
_S = 50
_NC, _NS = 2, 16
_NW = _NC * _NS
_BPW = _B // _NW
_NBUF = 2


def _sc_body(idx_hbm, zeros_hbm, out_hbm, idx_v, buf0, buf1, sem0, sem1):
    wid = lax.axis_index("s") * _NC + lax.axis_index("c")
    base_row = wid * _BPW * _S
    pltpu.sync_copy(idx_hbm.at[pl.ds(base_row, _BPW * _S)], idx_v)
    pltpu.sync_copy(zeros_hbm, buf0)
    pltpu.sync_copy(zeros_hbm, buf1)

    zeros16 = jnp.zeros((16,), jnp.float32)
    ones16 = jnp.ones((16,), jnp.float32)
    lane = lax.iota(jnp.int32, 16)
    tail_mask = lane >= 16 - (_S % 16)
    sems = (sem0, sem1)
    bufs = (buf0, buf1)

    def scatter_chunk(b, c, vals):
        off = c * _S
        for g in range(_S // 16):
            iv = idx_v[pl.ds(off + g * 16, 16)]
            plsc.store_scatter(bufs[b], [lane + g * 16, iv], vals)
        iv = idx_v[pl.ds(off + _S - 16, 16)]
        plsc.store_scatter(bufs[b], [lane + _S - 16, iv], vals,
                           mask=tail_mask)

    def fire(b, c):
        pltpu.async_copy(bufs[b], out_hbm.at[wid * _BPW + c], sems[b])

    def wait(b):
        pltpu.make_async_copy(bufs[b], out_hbm.at[0], sems[b]).wait()

    for b in range(_NBUF):
        scatter_chunk(b, b, ones16)
        fire(b, b)

    def ring_body(c, carry):
        def step(b):
            wait(b)
            scatter_chunk(b, c - _NBUF, zeros16)
            scatter_chunk(b, c, ones16)
            fire(b, c)

        @pl.when(lax.rem(c, 2) == 0)
        def _():
            step(0)

        @pl.when(lax.rem(c, 2) == 1)
        def _():
            step(1)
        return carry
    lax.fori_loop(_NBUF, _BPW, ring_body, 0)

    for b in range(_NBUF):
        wait(b)


def _sc_one_hot(flat_idx_i32, zeros_slab):
    mesh = plsc.VectorSubcoreMesh(core_axis_name="c", subcore_axis_name="s")
    k = functools.partial(
        pl.kernel, mesh=mesh,
        compiler_params=pltpu.CompilerParams(needs_layout_passes=False),
        out_type=jax.ShapeDtypeStruct((_B, _S, _N), jnp.float32),
        scratch_types=[
            pltpu.VMEM((_BPW * _S,), jnp.int32),
            pltpu.VMEM((_S, _N), jnp.float32),
            pltpu.VMEM((_S, _N), jnp.float32),
            pltpu.SemaphoreType.DMA,
            pltpu.SemaphoreType.DMA,
        ],
    )(_sc_body)
    return k(flat_idx_i32, zeros_slab)


def kernel(input, eye):
    del eye
    flat = input.reshape(-1).astype(jnp.int32)
    zeros_slab = jnp.zeros((_S, _N), jnp.float32)
    return _sc_one_hot(flat, zeros_slab)


# speedup vs baseline: 1.9313x; 1.0423x over previous
import functools
import jax
import jax.numpy as jnp
from jax import lax
from jax.experimental import pallas as pl
from jax.experimental.pallas import tpu as pltpu, tpu_sc as plsc

_N = 1000
_B = 1024
_S = 50
_NC, _NS = 2, 16
_NW = _NC * _NS
_BPW = _B // _NW
_NBUF = 2
_TAIL = _S % 16


def _sc_body(idx_hbm, out_hbm, idx_v, buf0, buf1, sem0, sem1):
    wid = lax.axis_index("s") * _NC + lax.axis_index("c")
    pltpu.sync_copy(idx_hbm.at[pl.ds(wid * _BPW, _BPW)], idx_v)

    zeros16 = jnp.zeros((16,), jnp.float32)
    ones16 = jnp.ones((16,), jnp.float32)
    lane = lax.iota(jnp.int32, 16)
    tail_mask = lane >= 16 - _TAIL
    sems = (sem0, sem1)
    bufs = (buf0, buf1)

    ztail_mask = lane >= 16 - (_N % 16)
    ztail_cols = lane + (_N - 16)

    def zrow(r, carry):
        for k in range(_N // 16):
            buf0[r, pl.ds(k * 16, 16)] = zeros16
            buf1[r, pl.ds(k * 16, 16)] = zeros16
        rr = jnp.broadcast_to(r, (16,)).astype(jnp.int32)
        plsc.store_scatter(buf0, [rr, ztail_cols], zeros16, mask=ztail_mask)
        plsc.store_scatter(buf1, [rr, ztail_cols], zeros16, mask=ztail_mask)
        return carry
    lax.fori_loop(0, _S, zrow, 0)

    def scatter_chunk(b, c, vals):
        for g in range(_S // 16):
            iv = idx_v[c, pl.ds(g * 16, 16)]
            plsc.store_scatter(bufs[b], [lane + g * 16, iv], vals)
        iv = idx_v[c, pl.ds(_S - 16, 16)]
        plsc.store_scatter(bufs[b], [lane + _S - 16, iv], vals,
                           mask=tail_mask)

    def fire(b, c):
        pltpu.async_copy(bufs[b], out_hbm.at[wid * _BPW + c], sems[b])

    def wait(b):
        pltpu.make_async_copy(bufs[b], out_hbm.at[0], sems[b]).wait()

    for b in range(_NBUF):
        scatter_chunk(b, b, ones16)
        fire(b, b)

    def ring_body(c, carry):
        def step(b):
            wait(b)
            scatter_chunk(b, c - _NBUF, zeros16)
            scatter_chunk(b, c, ones16)
            fire(b, c)

        @pl.when(lax.rem(c, 2) == 0)
        def _():
            step(0)

        @pl.when(lax.rem(c, 2) == 1)
        def _():
            step(1)
        return carry
    lax.fori_loop(_NBUF, _BPW, ring_body, 0)

    for b in range(_NBUF):
        wait(b)


def _sc_one_hot(idx_2d_i32):
    mesh = plsc.VectorSubcoreMesh(core_axis_name="c", subcore_axis_name="s")
    k = functools.partial(
        pl.kernel, mesh=mesh,
        compiler_params=pltpu.CompilerParams(needs_layout_passes=False),
        out_type=jax.ShapeDtypeStruct((_B, _S, _N), jnp.float32),
        scratch_types=[
            pltpu.VMEM((_BPW, _S), jnp.int32),
            pltpu.VMEM((_S, _N), jnp.float32),
            pltpu.VMEM((_S, _N), jnp.float32),
            pltpu.SemaphoreType.DMA,
            pltpu.SemaphoreType.DMA,
        ],
    )(_sc_body)
    return k(idx_2d_i32)


def kernel(input, eye):
    del eye
    return _sc_one_hot(input.astype(jnp.int32))


# speedup vs baseline: 4.1308x; 2.1389x over previous
import functools
import jax
import jax.numpy as jnp
from jax import lax
from jax.experimental import pallas as pl
from jax.experimental.pallas import tpu as pltpu, tpu_sc as plsc

_N = 1000
_B = 1024
_S = 50
_NC, _NS = 2, 16
_NW = _NC * _NS
_CN = 40
_NJ = _N // _CN
_NU = _S * _NJ
_NBUF = 2


def _sc_body(idxt_hbm, out_hbm, row0, row1, buf0, buf1, sem0, sem1):
    wid = lax.axis_index("s") * _NC + lax.axis_index("c")

    zeros16 = jnp.zeros((16,), jnp.float32)
    ones16 = jnp.ones((16,), jnp.float32)
    lane = lax.iota(jnp.int32, 16)
    sems = (sem0, sem1)
    bufs = (buf0, buf1)
    rows = (row0, row1)

    def zrow(r, carry):
        for k in range(_B // 16):
            buf0[r, pl.ds(k * 16, 16)] = zeros16
            buf1[r, pl.ds(k * 16, 16)] = zeros16
        return carry
    lax.fori_loop(0, _CN, zrow, 0)

    n_units = (_NU - 1 - wid) // _NW + 1

    def scatter_unit(b, n0, vals):
        for g in range(_B // 16):
            bl = lane + g * 16
            iv = rows[b][pl.ds(g * 16, 16)]
            m = (iv >= n0) & (iv < n0 + _CN)
            plsc.store_scatter(bufs[b], [iv - n0, bl], vals, mask=m)

    def load_unit(b, s):
        pltpu.sync_copy(idxt_hbm.at[s], rows[b])

    def fire(b, s, n0):
        dst = out_hbm.at[s, pl.ds(n0, _CN), :]
        pltpu.async_copy(bufs[b], dst, sems[b])

    def wait(b):
        pltpu.make_async_copy(
            bufs[b], out_hbm.at[0, pl.ds(0, _CN), :], sems[b]
        ).wait()

    def unit_of(c):
        u = wid + c * _NW
        return u // _NJ, lax.rem(u, _NJ) * _CN

    for b in range(_NBUF):
        s, n0 = unit_of(jnp.int32(b))
        load_unit(b, s)
        scatter_unit(b, n0, ones16)
        fire(b, s, n0)

    def ring_body(c, carry):
        def step(b):
            s, n0 = unit_of(c)
            _, n0_old = unit_of(c - _NBUF)
            wait(b)
            scatter_unit(b, n0_old, zeros16)
            load_unit(b, s)
            scatter_unit(b, n0, ones16)
            fire(b, s, n0)

        @pl.when(lax.rem(c, 2) == 0)
        def _():
            step(0)

        @pl.when(lax.rem(c, 2) == 1)
        def _():
            step(1)
        return carry
    lax.fori_loop(_NBUF, n_units, ring_body, 0)

    for b in range(_NBUF):
        wait(b)


def _sc_one_hot(idx_t_i32):
    mesh = plsc.VectorSubcoreMesh(core_axis_name="c", subcore_axis_name="s")
    k = functools.partial(
        pl.kernel, mesh=mesh,
        compiler_params=pltpu.CompilerParams(needs_layout_passes=False),
        out_type=jax.ShapeDtypeStruct((_S, _N, _B), jnp.float32),
        scratch_types=[
            pltpu.VMEM((_B,), jnp.int32),
            pltpu.VMEM((_B,), jnp.int32),
            pltpu.VMEM((_CN, _B), jnp.float32),
            pltpu.VMEM((_CN, _B), jnp.float32),
            pltpu.SemaphoreType.DMA,
            pltpu.SemaphoreType.DMA,
        ],
    )(_sc_body)
    return k(idx_t_i32)


def kernel(input, eye):
    del eye
    idx_t = jnp.transpose(input.astype(jnp.int32))
    out3 = _sc_one_hot(idx_t)
    return jnp.transpose(out3, (2, 0, 1))
